# Initial kernel scaffold; baseline (speedup 1.0000x reference)
import jax
import jax.numpy as jnp
from jax.experimental import pallas as pl


def kernel(x, neigh, W_in, b_in, Wc0, bc0, Wc1, bc1, Wc2, bc2, Wr1, br1, Wr2, br2):
    raise NotImplementedError("write your pallas kernel here")


# speedup vs baseline: 2.0701x; 2.0701x over previous
import functools

import jax
import jax.numpy as jnp
from jax import lax
from jax.experimental import pallas as pl
from jax.experimental.pallas import tpu as pltpu
from jax.experimental.pallas import tpu_sc as plsc

B, N, D, H, C, MAXDEG = 2, 10000, 128, 128, 16, 16
NPAD = 10240
TN = 256
GRID = NPAD // TN
NC, NS = 2, 16
NWORK = NC * NS
HALF = NPAD // NC
TILE_NODES = HALF // NS
NSUB = 5
SUBSZ = TILE_NODES // NSUB


def _proj_body(x_ref, neigh_ref, win_ref, bin_ref, wc_ref, bc_ref,
               g_ref, s_ref, inv_ref):
    nt = pl.program_id(0)
    rows = lax.broadcasted_iota(jnp.int32, (TN, 1), 0) + nt * TN
    valid = rows < N
    cnt = jnp.sum((neigh_ref[...] >= 0).astype(jnp.float32), axis=1,
                  keepdims=True)
    inv_ref[...] = jnp.broadcast_to(1.0 / jnp.maximum(cnt, 1.0), (TN, H))
    for b in range(B):
        h = jnp.dot(x_ref[b], win_ref[...],
                    preferred_element_type=jnp.float32) + bin_ref[...]
        g = jnp.dot(h, wc_ref[H:, :], preferred_element_type=jnp.float32)
        g_ref[:, b * H:(b + 1) * H] = jnp.where(valid, g, 0.0)
        s_ref[b] = jnp.dot(h, wc_ref[:H, :],
                           preferred_element_type=jnp.float32) + bc_ref[...]


def _comb_body(s_in_ref, agg_ref, inv_ref, wc_ref, bc_ref, g_ref, s_out_ref):
    nt = pl.program_id(0)
    rows = lax.broadcasted_iota(jnp.int32, (TN, 1), 0) + nt * TN
    valid = rows < N
    inv = inv_ref[...]
    for b in range(B):
        h = jnp.maximum(
            s_in_ref[b] + agg_ref[:, b * H:(b + 1) * H] * inv, 0.0)
        g = jnp.dot(h, wc_ref[H:, :], preferred_element_type=jnp.float32)
        g_ref[:, b * H:(b + 1) * H] = jnp.where(valid, g, 0.0)
        s_out_ref[b] = jnp.dot(h, wc_ref[:H, :],
                               preferred_element_type=jnp.float32) + bc_ref[...]


def _final_body(s_in_ref, agg_ref, inv_ref, out_ref):
    nt = pl.program_id(0)
    rows = lax.broadcasted_iota(jnp.int32, (TN, 1), 0) + nt * TN
    valid = rows < N
    inv = inv_ref[...]
    parts = []
    for b in range(B):
        h = jnp.maximum(
            s_in_ref[b] + agg_ref[:, b * H:(b + 1) * H] * inv, 0.0)
        h = jnp.where(valid, h, 0.0)
        parts.append(jnp.sum(h, axis=0)[None, :])
    partial = jnp.concatenate(parts, axis=0)

    @pl.when(nt == 0)
    def _():
        out_ref[...] = partial

    @pl.when(nt != 0)
    def _():
        out_ref[...] = out_ref[...] + partial


def _mlp_body(g_ref, wr1_ref, br1_ref, wr2_ref, br2_ref, out_ref):
    t = jnp.maximum(jnp.dot(g_ref[...], wr1_ref[...],
                            preferred_element_type=jnp.float32)
                    + br1_ref[...], 0.0)
    out_ref[...] = jnp.dot(t, wr2_ref[...],
                           preferred_element_type=jnp.float32) + br2_ref[...]


_proj_call = pl.pallas_call(
    _proj_body,
    grid=(GRID,),
    in_specs=[
        pl.BlockSpec((B, TN, D), lambda i: (0, i, 0)),
        pl.BlockSpec((TN, MAXDEG), lambda i: (i, 0)),
        pl.BlockSpec((D, H), lambda i: (0, 0)),
        pl.BlockSpec((1, H), lambda i: (0, 0)),
        pl.BlockSpec((2 * H, H), lambda i: (0, 0)),
        pl.BlockSpec((1, H), lambda i: (0, 0)),
    ],
    out_specs=[
        pl.BlockSpec((TN, B * H), lambda i: (i, 0)),
        pl.BlockSpec((B, TN, H), lambda i: (0, i, 0)),
        pl.BlockSpec((TN, H), lambda i: (i, 0)),
    ],
    out_shape=[
        jax.ShapeDtypeStruct((NPAD, B * H), jnp.float32),
        jax.ShapeDtypeStruct((B, NPAD, H), jnp.float32),
        jax.ShapeDtypeStruct((NPAD, H), jnp.float32),
    ],
)

_comb_call = pl.pallas_call(
    _comb_body,
    grid=(GRID,),
    in_specs=[
        pl.BlockSpec((B, TN, H), lambda i: (0, i, 0)),
        pl.BlockSpec((TN, B * H), lambda i: (i, 0)),
        pl.BlockSpec((TN, H), lambda i: (i, 0)),
        pl.BlockSpec((2 * H, H), lambda i: (0, 0)),
        pl.BlockSpec((1, H), lambda i: (0, 0)),
    ],
    out_specs=[
        pl.BlockSpec((TN, B * H), lambda i: (i, 0)),
        pl.BlockSpec((B, TN, H), lambda i: (0, i, 0)),
    ],
    out_shape=[
        jax.ShapeDtypeStruct((NPAD, B * H), jnp.float32),
        jax.ShapeDtypeStruct((B, NPAD, H), jnp.float32),
    ],
)

_final_call = pl.pallas_call(
    _final_body,
    grid=(GRID,),
    in_specs=[
        pl.BlockSpec((B, TN, H), lambda i: (0, i, 0)),
        pl.BlockSpec((TN, B * H), lambda i: (i, 0)),
        pl.BlockSpec((TN, H), lambda i: (i, 0)),
    ],
    out_specs=pl.BlockSpec((B, H), lambda i: (0, 0)),
    out_shape=jax.ShapeDtypeStruct((B, H), jnp.float32),
)

_mlp_call = pl.pallas_call(
    _mlp_body,
    out_shape=jax.ShapeDtypeStruct((B, C), jnp.float32),
)


CHN = 8
EDG = CHN * MAXDEG
NCHUNK = TILE_NODES // CHN
LANES = 16
NCB = (B * H) // LANES


def _tree_sum(vals):
    while len(vals) > 1:
        nxt = [vals[i] + vals[i + 1] for i in range(0, len(vals) - 1, 2)]
        if len(vals) % 2:
            nxt.append(vals[-1])
        vals = nxt
    return vals[0]


def _sc_agg_body(table_hbm, idx_hbm, out_hbm, idx_v, buf0, buf1, outb,
                 gs0, gs1):
    c = lax.axis_index("c")
    s = lax.axis_index("s")
    wid = c * NS + s
    base = wid * TILE_NODES
    pltpu.sync_copy(idx_hbm.at[wid], idx_v)

    def reduce_chunk(buf, ch):
        def jbody(j, carry):
            for cb in range(NCB):
                cols = pl.ds(cb * LANES, LANES)
                outb[j, cols] = _tree_sum(
                    [buf[j * MAXDEG + r, cols] for r in range(MAXDEG)])
            return carry
        lax.fori_loop(0, CHN, jbody, 0)
        pltpu.sync_copy(outb, out_hbm.at[pl.ds(base + ch * CHN, CHN)])

    pltpu.async_copy(table_hbm.at[idx_v.at[0]], buf0, gs0)

    def body(p, carry):
        ch0 = 2 * p
        pltpu.async_copy(table_hbm.at[idx_v.at[ch0 + 1]], buf1, gs1)
        pltpu.make_async_copy(table_hbm.at[idx_v.at[ch0]], buf0, gs0).wait()
        reduce_chunk(buf0, ch0)

        @pl.when(p < NCHUNK // 2 - 1)
        def _():
            pltpu.async_copy(table_hbm.at[idx_v.at[ch0 + 2]], buf0, gs0)

        pltpu.make_async_copy(table_hbm.at[idx_v.at[ch0 + 1]], buf1,
                              gs1).wait()
        reduce_chunk(buf1, ch0 + 1)
        return carry

    lax.fori_loop(0, NCHUNK // 2, body, 0)


@functools.lru_cache(maxsize=None)
def _sc_agg_call():
    return functools.partial(
        pl.kernel,
        out_type=jax.ShapeDtypeStruct((NPAD, B * H), jnp.float32),
        mesh=plsc.VectorSubcoreMesh(core_axis_name="c", subcore_axis_name="s",
                                    num_cores=NC, num_subcores=NS),
        scratch_types=(
            [pltpu.VMEM((NCHUNK, EDG), jnp.int32)]
            + [pltpu.VMEM((EDG, B * H), jnp.float32) for _ in range(2)]
            + [pltpu.VMEM((CHN, B * H), jnp.float32)]
            + [pltpu.SemaphoreType.DMA for _ in range(2)]
        ),
    )(_sc_agg_body)


def _sc_agg(table, idx4):
    return _sc_agg_call()(table, idx4)


def kernel(x, neigh, W_in, b_in, Wc0, bc0, Wc1, bc1, Wc2, bc2, Wr1, br1,
           Wr2, br2):
    neigh_i = neigh.astype(jnp.int32)
    idxc = jnp.where(neigh_i >= 0, neigh_i, N)
    idx_pad = jnp.concatenate(
        [idxc, jnp.full((NPAD - N, MAXDEG), N, jnp.int32)], axis=0)
    idx4 = idx_pad.reshape(NWORK, NCHUNK, EDG)
    neigh_pad = jnp.concatenate(
        [neigh_i, jnp.full((NPAD - N, MAXDEG), -1, jnp.int32)], axis=0)
    x_pad = jnp.concatenate(
        [x, jnp.zeros((B, NPAD - N, D), jnp.float32)], axis=1)

    g0, s0, inv = _proj_call(x_pad, neigh_pad, W_in, b_in.reshape(1, H),
                             Wc0, bc0.reshape(1, H))
    agg0 = _sc_agg(g0, idx4)
    g1, s1 = _comb_call(s0, agg0, inv, Wc1, bc1.reshape(1, H))
    agg1 = _sc_agg(g1, idx4)
    g2, s2 = _comb_call(s1, agg1, inv, Wc2, bc2.reshape(1, H))
    agg2 = _sc_agg(g2, idx4)
    gsum = _final_call(s2, agg2, inv)
    return _mlp_call(gsum, Wr1, br1.reshape(1, H), Wr2, br2.reshape(1, C))
